# baseline (speedup 1.0000x reference)
import functools

import jax
import jax.numpy as jnp
from jax import lax
from jax.experimental import pallas as pl
from jax.experimental.pallas import tpu as pltpu
from jax.experimental.pallas import tpu_sc as plsc

N = 10000
E = 320000
C = 128
NC = 2
NS = 16
NW = NC * NS
NPAD = 10240
RPW = NPAD // NS
CHUNK = 80
EPW = E // NW
NCH = EPW // CHUNK
DPW = E // NS
DCH = DPW // CHUNK
C2 = C // 2

_MESH = dict(core_axis_name="c", subcore_axis_name="s")


def _zero_1d(ref, n):
    @pl.loop(0, n, step=16)
    def _(i):
        ref[pl.ds(i, 16)] = jnp.zeros((16,), jnp.float32)


def _zero_2d(ref, n):
    @pl.loop(0, n)
    def _(r):
        @pl.loop(0, C, step=16)
        def _(j):
            ref[r, pl.ds(j, 16)] = jnp.zeros((16,), jnp.float32)


def _sc_hist(dst_r):
    @functools.partial(
        pl.kernel,
        out_type=[jax.ShapeDtypeStruct((NPAD,), jnp.float32)] * NC,
        mesh=plsc.VectorSubcoreMesh(**_MESH),
        scratch_types=[
            pltpu.VMEM((NCH, CHUNK), jnp.int32),
            pltpu.VMEM((CHUNK,), jnp.float32),
            pltpu.VMEM((RPW,), jnp.float32),
            pltpu.VMEM_SHARED((NPAD,), jnp.float32),
            pltpu.SemaphoreType.DMA,
        ],
    )
    def k(dst_hbm, out0_hbm, out1_hbm, idx_v, ones_v, zbuf_v, acc_s, sem):
        cid = lax.axis_index("c")
        sid = lax.axis_index("s")
        wid = cid * NS + sid

        pltpu.sync_copy(dst_hbm.at[wid], idx_v)

        @pl.loop(0, CHUNK, step=16)
        def _(i):
            ones_v[pl.ds(i, 16)] = jnp.full((16,), 1.0, jnp.float32)

        _zero_1d(zbuf_v, RPW)
        pltpu.sync_copy(zbuf_v, acc_s.at[pl.ds(sid * RPW, RPW)])
        plsc.subcore_barrier()

        @pl.loop(0, NCH)
        def _(ci):
            pltpu.sync_copy(ones_v, acc_s.at[idx_v.at[ci]], add=True)

        plsc.subcore_barrier()

        @pl.when(cid == 0)
        def _():
            pltpu.sync_copy(acc_s.at[pl.ds(sid * RPW, RPW)],
                            out0_hbm.at[pl.ds(sid * RPW, RPW)])

        @pl.when(cid == 1)
        def _():
            pltpu.sync_copy(acc_s.at[pl.ds(sid * RPW, RPW)],
                            out1_hbm.at[pl.ds(sid * RPW, RPW)])

    return k(dst_r)


_NBUF = 5
_ANB = 4
_NGRP = 31


def _sc_agg(pgrp, st, dt, y):
    @functools.partial(
        pl.kernel,
        out_type=[jax.ShapeDtypeStruct((NPAD, C), jnp.float32)] * NC,
        mesh=plsc.VectorSubcoreMesh(**_MESH),
        scratch_types=[
            pltpu.VMEM((2 * _ANB, CHUNK), jnp.int32),
            pltpu.VMEM((2 * _ANB, CHUNK), jnp.int32),
            pltpu.VMEM((1, CHUNK), jnp.int32),
            pltpu.VMEM((1, CHUNK), jnp.int32),
            pltpu.VMEM((_ANB, CHUNK, C), jnp.float32),
            pltpu.VMEM_SHARED((NPAD, C), jnp.float32),
            pltpu.SemaphoreType.DMA((_ANB,)),
        ],
    )
    def k(p_hbm, st_hbm, dt_hbm, y_hbm, out0_hbm, out1_hbm,
          ixa, ixb, tsx, tdx, rows_v, acc_s, sg):
        cid = lax.axis_index("c")
        sid = lax.axis_index("s")
        wid = cid * NS + sid

        _zero_2d(rows_v.at[0], CHUNK)

        @pl.loop(0, RPW // CHUNK)
        def _(t):
            pltpu.sync_copy(rows_v.at[0],
                            acc_s.at[pl.ds(sid * RPW + t * CHUNK, CHUNK)])

        plsc.subcore_barrier()

        pltpu.sync_copy(p_hbm.at[wid, 0], ixa)
        for b in range(_ANB):
            pltpu.async_copy(y_hbm.at[ixa.at[b]], rows_v.at[b], sg.at[b])

        def halfstep(gcur, cur, nxt):
            pltpu.sync_copy(p_hbm.at[wid, gcur + 1], nxt)
            for b in range(_ANB):
                pltpu.make_async_copy(y_hbm.at[cur.at[b]],
                                      rows_v.at[b], sg.at[b]).wait()
                pltpu.sync_copy(rows_v.at[b], acc_s.at[cur.at[_ANB + b]],
                                add=True)
                pltpu.async_copy(y_hbm.at[nxt.at[b]], rows_v.at[b], sg.at[b])

        @pl.loop(0, (_NGRP - 1) // 2)
        def _(t):
            halfstep(2 * t, ixa, ixb)
            halfstep(2 * t + 1, ixb, ixa)

        for b in range(_ANB):
            pltpu.make_async_copy(y_hbm.at[ixa.at[b]],
                                  rows_v.at[b], sg.at[b]).wait()
            pltpu.sync_copy(rows_v.at[b], acc_s.at[ixa.at[_ANB + b]], add=True)

        pltpu.sync_copy(st_hbm.at[wid], tsx)
        pltpu.sync_copy(dt_hbm.at[wid], tdx)
        pltpu.async_copy(y_hbm.at[tsx.at[0]], rows_v.at[0], sg.at[0]).wait()
        pltpu.sync_copy(rows_v.at[0], acc_s.at[tdx.at[0]], add=True)

        plsc.subcore_barrier()

        @pl.when(cid == 0)
        def _():
            pltpu.sync_copy(acc_s.at[pl.ds(sid * RPW, RPW)],
                            out0_hbm.at[pl.ds(sid * RPW, RPW)])

        @pl.when(cid == 1)
        def _():
            pltpu.sync_copy(acc_s.at[pl.ds(sid * RPW, RPW)],
                            out1_hbm.at[pl.ds(sid * RPW, RPW)])

    return k(pgrp, st, dt, y)


_ECH = 125


def _sc_scores(e0, e1, g1d):
    @functools.partial(
        pl.kernel,
        out_type=jax.ShapeDtypeStruct((E,), jnp.float32),
        mesh=plsc.VectorSubcoreMesh(**_MESH),
        scratch_types=[
            pltpu.VMEM((EPW,), jnp.int32),
            pltpu.VMEM((EPW,), jnp.int32),
            pltpu.VMEM((_NBUF, CHUNK), jnp.int32),
            pltpu.VMEM((EPW,), jnp.float32),
            pltpu.SemaphoreType.DMA((_NBUF,)),
        ],
    )
    def k(e0_hbm, e1_hbm, g_hbm, out_hbm, ea_v, eb_v, cix, sc_v, sg):
        cid = lax.axis_index("c")
        sid = lax.axis_index("s")
        wid = cid * NS + sid
        base = wid * EPW

        pltpu.sync_copy(e0_hbm.at[pl.ds(base, EPW)], ea_v)
        pltpu.sync_copy(e1_hbm.at[pl.ds(base, EPW)], eb_v)

        def compute_idx(c, b):
            for j in range(0, CHUNK, 16):
                av = ea_v[pl.ds(c * CHUNK + j, 16)]
                bv = eb_v[pl.ds(c * CHUNK + j, 16)]
                slab = jax.lax.shift_right_logical(bv, 7)
                lane = jax.lax.bitwise_and(bv, 127)
                cix[b, pl.ds(j, 16)] = slab * (N * C) + av * C + lane

        for b in range(_NBUF):
            compute_idx(b, b)
            pltpu.async_copy(g_hbm.at[cix.at[b]],
                             sc_v.at[pl.ds(b * CHUNK, CHUNK)], sg.at[b])

        @pl.loop(0, _ECH // _NBUF)
        def _(g):
            for b in range(_NBUF):
                c = g * _NBUF + b
                pltpu.make_async_copy(
                    g_hbm.at[cix.at[b]],
                    sc_v.at[pl.ds(c * CHUNK, CHUNK)], sg.at[b]).wait()

                @pl.when(g < _ECH // _NBUF - 1)
                def _():
                    compute_idx(c + _NBUF, b)
                    pltpu.async_copy(
                        g_hbm.at[cix.at[b]],
                        sc_v.at[pl.ds((c + _NBUF) * CHUNK, CHUNK)], sg.at[b])

        pltpu.sync_copy(sc_v, out_hbm.at[pl.ds(base, EPW)])

    return k(e0, e1, g1d)


_BM = 2000


def _tc_mm(x, W):
    def body(x_ref, w_ref, o_ref):
        o_ref[...] = jnp.dot(x_ref[...], w_ref[...],
                             preferred_element_type=jnp.float32,
                             precision=lax.Precision.HIGHEST)

    return pl.pallas_call(
        body,
        grid=(N // _BM,),
        in_specs=[
            pl.BlockSpec((_BM, C), lambda i: (i, 0)),
            pl.BlockSpec((C, C), lambda i: (0, 0)),
        ],
        out_specs=pl.BlockSpec((_BM, C), lambda i: (i, 0)),
        out_shape=jax.ShapeDtypeStruct((N, C), jnp.float32),
    )(x, W)


def _tc_scale(xw, h0, h1):
    def body(xw_ref, h0_ref, h1_ref, y_ref, d_ref):
        d = lax.rsqrt(h0_ref[...] + h1_ref[...] + 1.0)
        y_ref[...] = d * xw_ref[...]
        d_ref[...] = d

    return pl.pallas_call(
        body,
        grid=(N // _BM,),
        in_specs=[
            pl.BlockSpec((_BM, C), lambda i: (i, 0)),
            pl.BlockSpec((_BM, 1), lambda i: (i, 0)),
            pl.BlockSpec((_BM, 1), lambda i: (i, 0)),
        ],
        out_specs=[
            pl.BlockSpec((_BM, C), lambda i: (i, 0)),
            pl.BlockSpec((_BM, 1), lambda i: (i, 0)),
        ],
        out_shape=[
            jax.ShapeDtypeStruct((N, C), jnp.float32),
            jax.ShapeDtypeStruct((N, 1), jnp.float32),
        ],
    )(xw, h0, h1)


def _tc_fused_mid(acc0, acc1, y1, dinv, b1, W2):
    def body(a0_ref, a1_ref, y_ref, d_ref, b_ref, w_ref, o_ref):
        d = d_ref[...]
        h = jnp.maximum(d * (a0_ref[...] + a1_ref[...] + y_ref[...]) + b_ref[...],
                        0.0)
        o_ref[...] = d * jnp.dot(h, w_ref[...],
                                 preferred_element_type=jnp.float32,
                                 precision=lax.Precision.HIGHEST)

    return pl.pallas_call(
        body,
        grid=(N // _BM,),
        in_specs=[
            pl.BlockSpec((_BM, C), lambda i: (i, 0)),
            pl.BlockSpec((_BM, C), lambda i: (i, 0)),
            pl.BlockSpec((_BM, C), lambda i: (i, 0)),
            pl.BlockSpec((_BM, 1), lambda i: (i, 0)),
            pl.BlockSpec((1, C), lambda i: (0, 0)),
            pl.BlockSpec((C, C), lambda i: (0, 0)),
        ],
        out_specs=pl.BlockSpec((_BM, C), lambda i: (i, 0)),
        out_shape=jax.ShapeDtypeStruct((N, C), jnp.float32),
    )(acc0, acc1, y1, dinv, b1, W2)


def _tc_final(acc0, acc1, y2, dinv, b2):
    def body(a0_ref, a1_ref, y_ref, d_ref, b_ref, o_ref):
        o_ref[...] = (d_ref[...] * (a0_ref[...] + a1_ref[...] + y_ref[...])
                      + b_ref[...]).astype(jnp.bfloat16)

    return pl.pallas_call(
        body,
        grid=(N // _BM,),
        in_specs=[
            pl.BlockSpec((_BM, C), lambda i: (i, 0)),
            pl.BlockSpec((_BM, C), lambda i: (i, 0)),
            pl.BlockSpec((_BM, C), lambda i: (i, 0)),
            pl.BlockSpec((_BM, 1), lambda i: (i, 0)),
            pl.BlockSpec((1, C), lambda i: (0, 0)),
        ],
        out_specs=pl.BlockSpec((_BM, C), lambda i: (i, 0)),
        out_shape=jax.ShapeDtypeStruct((N, C), jnp.bfloat16),
    )(acc0, acc1, y2, dinv, b2)


_GBM = 2000
_NSLAB = 80
_SPB = 16


def _tc_gram(z16, zp16):
    def body(a_ref, zf_ref, o_ref):
        jb = pl.program_id(1)
        a = a_ref[...]
        for jj in range(_SPB // 2):
            zs = zf_ref[pl.ds((jb * _SPB + 2 * jj) * 128, 256), :]
            r = jax.lax.dot_general(
                a, zs, (((1,), (1,)), ((), ())),
                preferred_element_type=jnp.float32)
            o_ref[2 * jj] = r[:, 0:128]
            o_ref[2 * jj + 1] = r[:, 128:256]

    return pl.pallas_call(
        body,
        grid=(N // _GBM, _NSLAB // _SPB),
        in_specs=[
            pl.BlockSpec((_GBM, C), lambda i, j: (i, 0)),
            pl.BlockSpec((_NSLAB * 128, C), lambda i, j: (0, 0)),
        ],
        out_specs=pl.BlockSpec((_SPB, _GBM, C), lambda i, j: (j, i, 0)),
        out_shape=jax.ShapeDtypeStruct((_NSLAB, N, C), jnp.float32),
    )(z16, zp16)


def kernel(x, edge_index, edge_label_index, W1, b1, W2, b2):
    ei = edge_index.astype(jnp.int32)
    eli = edge_label_index.astype(jnp.int32)
    s_r = ei[0].reshape(NW, NCH, CHUNK)
    d_r = ei[1].reshape(NW, NCH, CHUNK)
    sgrp = s_r[:, : _NGRP * _ANB].reshape(NW, _NGRP, _ANB, CHUNK)
    dgrp = d_r[:, : _NGRP * _ANB].reshape(NW, _NGRP, _ANB, CHUNK)
    pgrp = jnp.concatenate([sgrp, dgrp], axis=2)
    st = s_r[:, _NGRP * _ANB :]
    dt = d_r[:, _NGRP * _ANB :]

    hist0, hist1 = _sc_hist(d_r)
    h0 = hist0[:, None]
    h1 = hist1[:, None]

    xw1 = _tc_mm(x, W1)
    y1, dinv = _tc_scale(xw1, h0, h1)
    a10, a11 = _sc_agg(pgrp, st, dt, y1)
    y2 = _tc_fused_mid(a10, a11, y1, dinv, b1.reshape(1, C), W2)
    a20, a21 = _sc_agg(pgrp, st, dt, y2)
    z16 = _tc_final(a20, a21, y2, dinv, b2.reshape(1, C))
    zp16 = jnp.pad(z16, ((0, _NSLAB * 128 - N), (0, 0)))
    g1d = _tc_gram(z16, zp16).reshape(_NSLAB * N * C)
    return _sc_scores(eli[0], eli[1], g1d)


# speedup vs baseline: 1.1273x; 1.1273x over previous
import dataclasses
import functools

import jax
import jax.numpy as jnp
from jax import lax
from jax.experimental import pallas as pl
from jax.experimental.pallas import tpu as pltpu
from jax.experimental.pallas import tpu_sc as plsc

N = 10000
E = 320000
C = 128
NC = 2
NS = 16
NW = NC * NS
NPAD = 10240
RPW = NPAD // NS
CHUNK = 80
EPW = E // NW
NCH = EPW // CHUNK
DPW = E // NS
DCH = DPW // CHUNK
C2 = C // 2

_MESH = dict(core_axis_name="c", subcore_axis_name="s")

_SC_CP = pltpu.CompilerParams()
if "needs_layout_passes" in pltpu.CompilerParams.__dataclass_fields__:
    _SC_CP = dataclasses.replace(_SC_CP, needs_layout_passes=False)


def _zero_1d(ref, n):
    @pl.loop(0, n, step=16)
    def _(i):
        ref[pl.ds(i, 16)] = jnp.zeros((16,), jnp.float32)


def _zero_2d(ref, n):
    @pl.loop(0, n)
    def _(r):
        @pl.loop(0, C, step=16)
        def _(j):
            ref[r, pl.ds(j, 16)] = jnp.zeros((16,), jnp.float32)


def _sc_hist(dst_r):
    @functools.partial(
        pl.kernel,
        out_type=[jax.ShapeDtypeStruct((NPAD,), jnp.float32)] * NC,
        mesh=plsc.VectorSubcoreMesh(**_MESH),
        scratch_types=[
            pltpu.VMEM((NCH, CHUNK), jnp.int32),
            pltpu.VMEM((CHUNK,), jnp.float32),
            pltpu.VMEM((RPW,), jnp.float32),
            pltpu.VMEM_SHARED((NPAD,), jnp.float32),
            pltpu.SemaphoreType.DMA,
        ],
    )
    def k(dst_hbm, out0_hbm, out1_hbm, idx_v, ones_v, zbuf_v, acc_s, sem):
        cid = lax.axis_index("c")
        sid = lax.axis_index("s")
        wid = cid * NS + sid

        pltpu.sync_copy(dst_hbm.at[wid], idx_v)

        @pl.loop(0, CHUNK, step=16)
        def _(i):
            ones_v[pl.ds(i, 16)] = jnp.full((16,), 1.0, jnp.float32)

        _zero_1d(zbuf_v, RPW)
        pltpu.sync_copy(zbuf_v, acc_s.at[pl.ds(sid * RPW, RPW)])
        plsc.subcore_barrier()

        @pl.loop(0, NCH)
        def _(ci):
            pltpu.sync_copy(ones_v, acc_s.at[idx_v.at[ci]], add=True)

        plsc.subcore_barrier()

        @pl.when(cid == 0)
        def _():
            pltpu.sync_copy(acc_s.at[pl.ds(sid * RPW, RPW)],
                            out0_hbm.at[pl.ds(sid * RPW, RPW)])

        @pl.when(cid == 1)
        def _():
            pltpu.sync_copy(acc_s.at[pl.ds(sid * RPW, RPW)],
                            out1_hbm.at[pl.ds(sid * RPW, RPW)])

    return k(dst_r)


_NBUF = 5
_ANB = 4
_NGRP = 31


def _sc_agg(pgrp, st, dt, y):
    @functools.partial(
        pl.kernel,
        out_type=[jax.ShapeDtypeStruct((NPAD, C), jnp.float32)] * NC,
        mesh=plsc.VectorSubcoreMesh(**_MESH),
        scratch_types=[
            pltpu.VMEM((2 * _ANB, CHUNK), jnp.int32),
            pltpu.VMEM((2 * _ANB, CHUNK), jnp.int32),
            pltpu.VMEM((1, CHUNK), jnp.int32),
            pltpu.VMEM((1, CHUNK), jnp.int32),
            pltpu.VMEM((_ANB, CHUNK, C), jnp.float32),
            pltpu.VMEM_SHARED((NPAD, C), jnp.float32),
            pltpu.SemaphoreType.DMA((_ANB,)),
        ],
    )
    def k(p_hbm, st_hbm, dt_hbm, y_hbm, out0_hbm, out1_hbm,
          ixa, ixb, tsx, tdx, rows_v, acc_s, sg):
        cid = lax.axis_index("c")
        sid = lax.axis_index("s")
        wid = cid * NS + sid

        _zero_2d(rows_v.at[0], CHUNK)

        @pl.loop(0, RPW // CHUNK)
        def _(t):
            pltpu.sync_copy(rows_v.at[0],
                            acc_s.at[pl.ds(sid * RPW + t * CHUNK, CHUNK)])

        plsc.subcore_barrier()

        pltpu.sync_copy(p_hbm.at[wid, 0], ixa)
        for b in range(_ANB):
            pltpu.async_copy(y_hbm.at[ixa.at[b]], rows_v.at[b], sg.at[b])

        def halfstep(gcur, cur, nxt):
            pltpu.sync_copy(p_hbm.at[wid, gcur + 1], nxt)
            for b in range(_ANB):
                pltpu.make_async_copy(y_hbm.at[cur.at[b]],
                                      rows_v.at[b], sg.at[b]).wait()
                pltpu.sync_copy(rows_v.at[b], acc_s.at[cur.at[_ANB + b]],
                                add=True)
                pltpu.async_copy(y_hbm.at[nxt.at[b]], rows_v.at[b], sg.at[b])

        @pl.loop(0, (_NGRP - 1) // 2)
        def _(t):
            halfstep(2 * t, ixa, ixb)
            halfstep(2 * t + 1, ixb, ixa)

        for b in range(_ANB):
            pltpu.make_async_copy(y_hbm.at[ixa.at[b]],
                                  rows_v.at[b], sg.at[b]).wait()
            pltpu.sync_copy(rows_v.at[b], acc_s.at[ixa.at[_ANB + b]], add=True)

        pltpu.sync_copy(st_hbm.at[wid], tsx)
        pltpu.sync_copy(dt_hbm.at[wid], tdx)
        pltpu.async_copy(y_hbm.at[tsx.at[0]], rows_v.at[0], sg.at[0]).wait()
        pltpu.sync_copy(rows_v.at[0], acc_s.at[tdx.at[0]], add=True)

        plsc.subcore_barrier()

        @pl.when(cid == 0)
        def _():
            pltpu.sync_copy(acc_s.at[pl.ds(sid * RPW, RPW)],
                            out0_hbm.at[pl.ds(sid * RPW, RPW)])

        @pl.when(cid == 1)
        def _():
            pltpu.sync_copy(acc_s.at[pl.ds(sid * RPW, RPW)],
                            out1_hbm.at[pl.ds(sid * RPW, RPW)])

    return k(pgrp, st, dt, y)


_ECH = 125


def _sc_scores(e0, e1, g1d):
    @functools.partial(
        pl.kernel,
        out_type=jax.ShapeDtypeStruct((E,), jnp.float32),
        mesh=plsc.VectorSubcoreMesh(**_MESH),
        compiler_params=_SC_CP,
        scratch_types=[
            pltpu.VMEM((EPW,), jnp.int32),
            pltpu.VMEM((EPW,), jnp.int32),
            pltpu.VMEM((_NBUF, CHUNK), jnp.int32),
            pltpu.VMEM((_NBUF, CHUNK), jnp.int32),
            pltpu.VMEM((EPW,), jnp.float32),
            pltpu.SemaphoreType.DMA((_NBUF,)),
        ],
    )
    def k(e0_hbm, e1_hbm, g_hbm, out_hbm, ea_v, eb_v, cix, gw_v, sc_v, sg):
        cid = lax.axis_index("c")
        sid = lax.axis_index("s")
        wid = cid * NS + sid
        base = wid * EPW

        pltpu.sync_copy(e0_hbm.at[pl.ds(base, EPW)], ea_v)
        pltpu.sync_copy(e1_hbm.at[pl.ds(base, EPW)], eb_v)

        def compute_idx(c, b):
            for j in range(0, CHUNK, 16):
                av = ea_v[pl.ds(c * CHUNK + j, 16)]
                bv = eb_v[pl.ds(c * CHUNK + j, 16)]
                slab = jax.lax.shift_right_logical(bv, 7)
                lane = jax.lax.bitwise_and(bv, 127)
                ah = jax.lax.shift_right_logical(av, 1)
                cix[b, pl.ds(j, 16)] = slab * (N * C // 2) + ah * C + lane

        def emit_scores(c, b):
            for j in range(0, CHUNK, 16):
                av = ea_v[pl.ds(c * CHUNK + j, 16)]
                gw = gw_v[b, pl.ds(j, 16)]
                odd = jax.lax.bitwise_and(av, 1) == 1
                hi = jax.lax.bitwise_and(gw, jnp.int32(-65536))
                lo = jax.lax.shift_left(gw, 16)
                bits = jnp.where(odd, hi, lo)
                sc_v[pl.ds(c * CHUNK + j, 16)] = plsc.bitcast(
                    bits, jnp.float32)

        for b in range(_NBUF):
            compute_idx(b, b)
            pltpu.async_copy(g_hbm.at[cix.at[b]], gw_v.at[b], sg.at[b])

        @pl.loop(0, _ECH // _NBUF)
        def _(g):
            for b in range(_NBUF):
                c = g * _NBUF + b
                pltpu.make_async_copy(
                    g_hbm.at[cix.at[b]], gw_v.at[b], sg.at[b]).wait()
                emit_scores(c, b)

                @pl.when(g < _ECH // _NBUF - 1)
                def _():
                    compute_idx(c + _NBUF, b)
                    pltpu.async_copy(g_hbm.at[cix.at[b]],
                                     gw_v.at[b], sg.at[b])

        pltpu.sync_copy(sc_v, out_hbm.at[pl.ds(base, EPW)])

    return k(e0, e1, g1d)


_BM = 2000


def _tc_mm(x, W):
    def body(x_ref, w_ref, o_ref):
        o_ref[...] = jnp.dot(x_ref[...], w_ref[...],
                             preferred_element_type=jnp.float32,
                             precision=lax.Precision.HIGHEST)

    return pl.pallas_call(
        body,
        grid=(N // _BM,),
        in_specs=[
            pl.BlockSpec((_BM, C), lambda i: (i, 0)),
            pl.BlockSpec((C, C), lambda i: (0, 0)),
        ],
        out_specs=pl.BlockSpec((_BM, C), lambda i: (i, 0)),
        out_shape=jax.ShapeDtypeStruct((N, C), jnp.float32),
    )(x, W)


def _tc_scale(xw, h0, h1):
    def body(xw_ref, h0_ref, h1_ref, y_ref, d_ref):
        d = lax.rsqrt(h0_ref[...] + h1_ref[...] + 1.0)
        y_ref[...] = d * xw_ref[...]
        d_ref[...] = d

    return pl.pallas_call(
        body,
        grid=(N // _BM,),
        in_specs=[
            pl.BlockSpec((_BM, C), lambda i: (i, 0)),
            pl.BlockSpec((_BM, 1), lambda i: (i, 0)),
            pl.BlockSpec((_BM, 1), lambda i: (i, 0)),
        ],
        out_specs=[
            pl.BlockSpec((_BM, C), lambda i: (i, 0)),
            pl.BlockSpec((_BM, 1), lambda i: (i, 0)),
        ],
        out_shape=[
            jax.ShapeDtypeStruct((N, C), jnp.float32),
            jax.ShapeDtypeStruct((N, 1), jnp.float32),
        ],
    )(xw, h0, h1)


def _tc_fused_mid(acc0, acc1, y1, dinv, b1, W2):
    def body(a0_ref, a1_ref, y_ref, d_ref, b_ref, w_ref, o_ref):
        d = d_ref[...]
        h = jnp.maximum(d * (a0_ref[...] + a1_ref[...] + y_ref[...]) + b_ref[...],
                        0.0)
        o_ref[...] = d * jnp.dot(h, w_ref[...],
                                 preferred_element_type=jnp.float32,
                                 precision=lax.Precision.HIGHEST)

    return pl.pallas_call(
        body,
        grid=(N // _BM,),
        in_specs=[
            pl.BlockSpec((_BM, C), lambda i: (i, 0)),
            pl.BlockSpec((_BM, C), lambda i: (i, 0)),
            pl.BlockSpec((_BM, C), lambda i: (i, 0)),
            pl.BlockSpec((_BM, 1), lambda i: (i, 0)),
            pl.BlockSpec((1, C), lambda i: (0, 0)),
            pl.BlockSpec((C, C), lambda i: (0, 0)),
        ],
        out_specs=pl.BlockSpec((_BM, C), lambda i: (i, 0)),
        out_shape=jax.ShapeDtypeStruct((N, C), jnp.float32),
    )(acc0, acc1, y1, dinv, b1, W2)


def _tc_final(acc0, acc1, y2, dinv, b2):
    def body(a0_ref, a1_ref, y_ref, d_ref, b_ref, o_ref):
        o_ref[...] = (d_ref[...] * (a0_ref[...] + a1_ref[...] + y_ref[...])
                      + b_ref[...]).astype(jnp.bfloat16)

    return pl.pallas_call(
        body,
        grid=(N // _BM,),
        in_specs=[
            pl.BlockSpec((_BM, C), lambda i: (i, 0)),
            pl.BlockSpec((_BM, C), lambda i: (i, 0)),
            pl.BlockSpec((_BM, C), lambda i: (i, 0)),
            pl.BlockSpec((_BM, 1), lambda i: (i, 0)),
            pl.BlockSpec((1, C), lambda i: (0, 0)),
        ],
        out_specs=pl.BlockSpec((_BM, C), lambda i: (i, 0)),
        out_shape=jax.ShapeDtypeStruct((N, C), jnp.bfloat16),
    )(acc0, acc1, y2, dinv, b2)


_GBM = 2000
_NSLAB = 80
_SPB = 16


def _tc_gram(z16, zp16):
    def body(a_ref, zf_ref, o_ref):
        jb = pl.program_id(1)
        a = a_ref[...]
        for jj in range(_SPB // 2):
            zs = zf_ref[pl.ds((jb * _SPB + 2 * jj) * 128, 256), :]
            r = jax.lax.dot_general(
                a, zs, (((1,), (1,)), ((), ())),
                preferred_element_type=jnp.float32)
            rp = pltpu.bitcast(r.astype(jnp.bfloat16), jnp.int32)
            o_ref[2 * jj] = rp[:, 0:128]
            o_ref[2 * jj + 1] = rp[:, 128:256]

    return pl.pallas_call(
        body,
        grid=(N // _GBM, _NSLAB // _SPB),
        in_specs=[
            pl.BlockSpec((_GBM, C), lambda i, j: (i, 0)),
            pl.BlockSpec((_NSLAB * 128, C), lambda i, j: (0, 0)),
        ],
        out_specs=pl.BlockSpec((_SPB, _GBM // 2, C), lambda i, j: (j, i, 0)),
        out_shape=jax.ShapeDtypeStruct((_NSLAB, N // 2, C), jnp.int32),
    )(z16, zp16)


def kernel(x, edge_index, edge_label_index, W1, b1, W2, b2):
    ei = edge_index.astype(jnp.int32)
    eli = edge_label_index.astype(jnp.int32)
    s_r = ei[0].reshape(NW, NCH, CHUNK)
    d_r = ei[1].reshape(NW, NCH, CHUNK)
    sgrp = s_r[:, : _NGRP * _ANB].reshape(NW, _NGRP, _ANB, CHUNK)
    dgrp = d_r[:, : _NGRP * _ANB].reshape(NW, _NGRP, _ANB, CHUNK)
    pgrp = jnp.concatenate([sgrp, dgrp], axis=2)
    st = s_r[:, _NGRP * _ANB :]
    dt = d_r[:, _NGRP * _ANB :]

    hist0, hist1 = _sc_hist(d_r)
    h0 = hist0[:, None]
    h1 = hist1[:, None]

    xw1 = _tc_mm(x, W1)
    y1, dinv = _tc_scale(xw1, h0, h1)
    a10, a11 = _sc_agg(pgrp, st, dt, y1)
    y2 = _tc_fused_mid(a10, a11, y1, dinv, b1.reshape(1, C), W2)
    a20, a21 = _sc_agg(pgrp, st, dt, y2)
    z16 = _tc_final(a20, a21, y2, dinv, b2.reshape(1, C))
    zp16 = jnp.pad(z16, ((0, _NSLAB * 128 - N), (0, 0)))
    g1d = _tc_gram(z16, zp16).reshape(_NSLAB * (N // 2) * C)
    return _sc_scores(eli[0], eli[1], g1d)
